# Initial kernel scaffold; baseline (speedup 1.0000x reference)
import jax
import jax.numpy as jnp
from jax.experimental import pallas as pl


def kernel(x, edge_index, batch, W1, b1, W2, b2, W3, b3, W4, b4, LW1, Lb1, LW2, Lb2):
    raise NotImplementedError("write your pallas kernel here")


# speedup vs baseline: 21.6764x; 21.6764x over previous
import functools

import jax
import jax.numpy as jnp
from jax import lax
from jax.experimental import pallas as pl
from jax.experimental.pallas import tpu as pltpu
from jax.experimental.pallas import tpu_sc as plsc

N = 10000
NP = 10240
E = 320000
G = 64
CHUNK = 128
K = 80
EC_ROWS = 2560
M = 8
H = M // 2
RPW = NP // 16
DW = 16

_SELU_L = 1.0507009873554805
_SELU_A = 1.6732632423543772


def _selu(x):
    return _SELU_L * jnp.where(x > 0, x, _SELU_A * (jnp.exp(x) - 1.0))


def _sc_mesh():
    return plsc.VectorSubcoreMesh(core_axis_name="c", subcore_axis_name="s")


_SC_PARAMS = pltpu.CompilerParams(use_tc_tiling_on_sc=False)


def _make_edge_agg(d):
    assert K % M == 0

    @functools.partial(
        pl.kernel,
        mesh=_sc_mesh(),
        compiler_params=_SC_PARAMS,
        out_type=jax.ShapeDtypeStruct((2, NP, d), jnp.float32),
        scratch_types=[
            pltpu.VMEM((K, CHUNK), jnp.int32),
            pltpu.VMEM((K, CHUNK), jnp.int32),
            pltpu.VMEM((M, CHUNK, d), jnp.float32),
            pltpu.VMEM_SHARED((NP, d), jnp.float32),
        ] + [pltpu.SemaphoreType.DMA] * (2 * M),
    )
    def agg(h_hbm, src_hbm, dst_hbm, out_hbm, sidx, didx, rows, acc, *sems):
        semg = sems[:M]
        sems_ = sems[M:]
        cid = lax.axis_index("c")
        sid = lax.axis_index("s")
        wid = cid * 16 + sid
        start = wid * K

        pltpu.sync_copy(src_hbm.at[pl.ds(start, K)], sidx)
        pltpu.sync_copy(dst_hbm.at[pl.ds(start, K)], didx)

        def zrow(r, carry):
            for jj in range(d // 16):
                rows[0, r, pl.ds(jj * 16, 16)] = jnp.zeros((16,), jnp.float32)
            return carry
        lax.fori_loop(0, CHUNK, zrow, 0)
        for t in range(RPW // CHUNK):
            pltpu.sync_copy(rows.at[0], acc.at[pl.ds(sid * RPW + t * CHUNK, CHUNK)])
        plsc.subcore_barrier()

        def gather(c, b):
            pltpu.async_copy(h_hbm.at[sidx.at[c]], rows.at[b], semg[b])

        def gather_wait(c, b):
            pltpu.make_async_copy(h_hbm.at[sidx.at[c]], rows.at[b], semg[b]).wait()

        def scat(c, b):
            pltpu.async_copy(rows.at[b], acc.at[didx.at[c]], sems_[b], add=True)

        def scat_wait(c, b):
            pltpu.make_async_copy(rows.at[b], acc.at[didx.at[c]], sems_[b]).wait()

        for i in range(H):
            gather(i, i)
        for b in range(H):
            gather_wait(b, b)
            scat(b, b)
            gather(b + H, b + H)

        def outer(gi, carry):
            c0 = H + gi * M
            for j in range(M):
                c = c0 + j
                b = (H + j) % M
                q = j
                gather_wait(c, b)
                scat(c, b)
                scat_wait(c - H, q)
                gather(c + H, q)
            return carry
        lax.fori_loop(0, (K - M) // M, outer, 0)

        for j in range(H):
            c = K - H + j
            b = c % M
            gather_wait(c, b)
            scat(c, b)
        for b in range(M):
            scat_wait(0, b)

        plsc.subcore_barrier()
        pltpu.sync_copy(acc.at[pl.ds(sid * RPW, RPW)],
                        out_hbm.at[cid, pl.ds(sid * RPW, RPW)])

    return agg


@functools.partial(
    pl.kernel,
    mesh=_sc_mesh(),
    compiler_params=_SC_PARAMS,
    out_type=jax.ShapeDtypeStruct((2, NP, DW), jnp.float32),
    scratch_types=[
        pltpu.VMEM((K, CHUNK), jnp.int32),
        pltpu.VMEM((CHUNK, DW), jnp.float32),
        pltpu.VMEM_SHARED((NP, DW), jnp.float32),
    ] + [pltpu.SemaphoreType.DMA] * M,
)
def _deg_count(dst_hbm, out_hbm, didx, ones, acc, *sems):
    cid = lax.axis_index("c")
    sid = lax.axis_index("s")
    wid = cid * 16 + sid
    start = wid * K

    pltpu.sync_copy(dst_hbm.at[pl.ds(start, K)], didx)

    def fill(val):
        def body(r, carry):
            ones[r, pl.ds(0, 16)] = jnp.full((16,), val, jnp.float32)
            return carry
        lax.fori_loop(0, CHUNK, body, 0)

    fill(0.0)
    for t in range(RPW // CHUNK):
        pltpu.sync_copy(ones.at[pl.ds(0, CHUNK)],
                        acc.at[pl.ds(sid * RPW + t * CHUNK, CHUNK)])
    plsc.subcore_barrier()
    fill(1.0)

    def scat(c, b):
        pltpu.async_copy(ones, acc.at[didx.at[c]], sems[b], add=True)

    def scat_wait(c, b):
        pltpu.make_async_copy(ones, acc.at[didx.at[c]], sems[b]).wait()

    for b in range(M):
        scat(b, b)

    def body(gi, carry):
        c0 = M + gi * M
        for j in range(M):
            scat_wait(c0 + j - M, j)
            scat(c0 + j, j)
        return carry
    lax.fori_loop(0, (K - M) // M, body, 0)
    for b in range(M):
        scat_wait(K - M + b, b)

    plsc.subcore_barrier()
    pltpu.sync_copy(acc.at[pl.ds(sid * RPW, RPW)],
                    out_hbm.at[cid, pl.ds(sid * RPW, RPW)])


_agg16 = _make_edge_agg(16)
_agg32 = _make_edge_agg(32)
_agg48 = _make_edge_agg(48)


def _tc_prep_body(xp_ref, w_ref, degp_ref, h1p_ref, dinv_ref):
    deg = 1.0 + degp_ref[0, :, 0:1] + degp_ref[1, :, 0:1]
    dinv = lax.rsqrt(deg)
    h1 = jnp.dot(xp_ref[...], w_ref[...], preferred_element_type=jnp.float32)
    h1p_ref[...] = h1 * dinv
    dinv_ref[...] = dinv


_tc_prep = pl.pallas_call(
    _tc_prep_body,
    out_shape=(jax.ShapeDtypeStruct((NP, 16), jnp.float32),
               jax.ShapeDtypeStruct((NP, 1), jnp.float32)),
)


def _tc_layer_body(parts_ref, hp_ref, dinv_ref, b_ref, w_ref, out_ref):
    dinv = dinv_ref[...]
    agg = (parts_ref[0] + parts_ref[1] + hp_ref[...]) * dinv + b_ref[...]
    c = _selu(agg)
    out_ref[...] = jnp.dot(c, w_ref[...], preferred_element_type=jnp.float32) * dinv


def _make_tc_layer(d_out):
    return pl.pallas_call(
        _tc_layer_body,
        out_shape=jax.ShapeDtypeStruct((NP, d_out), jnp.float32))


_tc_layer2 = _make_tc_layer(32)
_tc_layer3 = _make_tc_layer(32)
_tc_layer4 = _make_tc_layer(48)


def _tc_final_body(parts_ref, hp_ref, dinv_ref, b_ref, batch_ref,
                   lw1_ref, lb1_ref, lw2_ref, lb2_ref, out_ref):
    agg = (parts_ref[0] + parts_ref[1] + hp_ref[...]) * dinv_ref[...] + b_ref[...]
    c4 = _selu(agg)
    seg = lax.broadcasted_iota(jnp.int32, (G, NP), 0)
    m = (batch_ref[...] == seg).astype(jnp.float32)
    s = jnp.dot(m, c4, preferred_element_type=jnp.float32)
    col = lax.broadcasted_iota(jnp.int32, (G, 48), 1)
    s = jnp.where(col < 36, s, -jnp.inf)
    s = s - jnp.max(s, axis=1, keepdims=True)
    e = jnp.exp(s)
    p = e / jnp.sum(e, axis=1, keepdims=True)
    h = jnp.dot(p, lw1_ref[...], preferred_element_type=jnp.float32) + lb1_ref[...]
    h = jnp.maximum(h, 0.0)
    o = jnp.dot(h, lw2_ref[...], preferred_element_type=jnp.float32) + lb2_ref[...]
    out_ref[...] = jnp.maximum(o, 0.0)


_tc_final = pl.pallas_call(
    _tc_final_body,
    out_shape=jax.ShapeDtypeStruct((G, 16), jnp.float32))


def kernel(x, edge_index, batch, W1, b1, W2, b2, W3, b3, W4, b4,
           LW1, Lb1, LW2, Lb2):
    f32 = jnp.float32
    src = edge_index[0].astype(jnp.int32)
    dst = edge_index[1].astype(jnp.int32)
    pad_e = EC_ROWS * CHUNK - E
    srcp = jnp.concatenate([src, jnp.full((pad_e,), N, jnp.int32)])
    dstp = jnp.concatenate([dst, jnp.full((pad_e,), N, jnp.int32)])
    srcp = srcp.reshape(EC_ROWS, CHUNK)
    dstp = dstp.reshape(EC_ROWS, CHUNK)
    xp = jnp.zeros((NP, 128), f32).at[:N].set(x)
    batchp = jnp.full((1, NP), G, jnp.int32).at[0, :N].set(batch)

    def padw(w, r, c):
        return jnp.zeros((r, c), f32).at[:w.shape[0], :w.shape[1]].set(w)

    W1p = padw(W1, 128, 16)
    W2p = padw(W2, 16, 32)
    W3p = padw(W3, 32, 32)
    W4p = padw(W4, 32, 48)
    b1p = padw(b1[None], 1, 16)
    b2p = padw(b2[None], 1, 32)
    b3p = padw(b3[None], 1, 32)
    b4p = padw(b4[None], 1, 48)
    LW1p = padw(LW1, 48, 96)
    lb1p = padw(Lb1[None], 1, 96)
    LW2p = padw(LW2, 96, 16)
    lb2p = padw(Lb2[None], 1, 16)

    degp = _deg_count(dstp)
    h1p, dinv = _tc_prep(xp, W1p, degp)
    p1 = _agg16(h1p, srcp, dstp)
    h2p = _tc_layer2(p1, h1p, dinv, b1p, W2p)
    p2 = _agg32(h2p, srcp, dstp)
    h3p = _tc_layer3(p2, h2p, dinv, b2p, W3p)
    p3 = _agg32(h3p, srcp, dstp)
    h4p = _tc_layer4(p3, h3p, dinv, b3p, W4p)
    p4 = _agg48(h4p, srcp, dstp)
    out = _tc_final(p4, h4p, dinv, b4p, batchp, LW1p, lb1p, LW2p, lb2p)
    return out[:, :10]


# speedup vs baseline: 27.7741x; 1.2813x over previous
import functools

import jax
import jax.numpy as jnp
from jax import lax
from jax.experimental import pallas as pl
from jax.experimental.pallas import tpu as pltpu
from jax.experimental.pallas import tpu_sc as plsc

N = 10000
NP = 10240
E = 320000
G = 64
CHUNK = 128
K = 80
EC_ROWS = 2560
M = 8
H = M // 2
RPW = NP // 16
DW = 16

_SELU_L = 1.0507009873554805
_SELU_A = 1.6732632423543772


def _selu(x):
    return _SELU_L * jnp.where(x > 0, x, _SELU_A * (jnp.exp(x) - 1.0))


def _sc_mesh():
    return plsc.VectorSubcoreMesh(core_axis_name="c", subcore_axis_name="s")


_SC_PARAMS = pltpu.CompilerParams(use_tc_tiling_on_sc=False)


def _make_edge_agg(d):
    assert K % M == 0

    @functools.partial(
        pl.kernel,
        mesh=_sc_mesh(),
        compiler_params=_SC_PARAMS,
        out_type=jax.ShapeDtypeStruct((2, NP, d), jnp.float32),
        scratch_types=[
            pltpu.VMEM((K, CHUNK), jnp.int32),
            pltpu.VMEM((K, CHUNK), jnp.int32),
            pltpu.VMEM((M, CHUNK, d), jnp.float32),
            pltpu.VMEM_SHARED((NP, d), jnp.float32),
        ] + [pltpu.SemaphoreType.DMA] * (2 * M),
    )
    def agg(h_hbm, src_hbm, dst_hbm, out_hbm, sidx, didx, rows, acc, *sems):
        semg = sems[:M]
        sems_ = sems[M:]
        cid = lax.axis_index("c")
        sid = lax.axis_index("s")
        wid = cid * 16 + sid
        start = wid * K

        pltpu.sync_copy(src_hbm.at[pl.ds(start, K)], sidx)
        pltpu.sync_copy(dst_hbm.at[pl.ds(start, K)], didx)

        def zrow(r, carry):
            for jj in range(d // 16):
                rows[0, r, pl.ds(jj * 16, 16)] = jnp.zeros((16,), jnp.float32)
            return carry
        lax.fori_loop(0, CHUNK, zrow, 0)
        for t in range(RPW // CHUNK):
            pltpu.sync_copy(rows.at[0], acc.at[pl.ds(sid * RPW + t * CHUNK, CHUNK)])
        plsc.subcore_barrier()

        def gather(c, b):
            pltpu.async_copy(h_hbm.at[sidx.at[c]], rows.at[b], semg[b])

        def gather_wait(c, b):
            pltpu.make_async_copy(h_hbm.at[sidx.at[c]], rows.at[b], semg[b]).wait()

        def scat(c, b):
            pltpu.async_copy(rows.at[b], acc.at[didx.at[c]], sems_[b], add=True)

        def scat_wait(c, b):
            pltpu.make_async_copy(rows.at[b], acc.at[didx.at[c]], sems_[b]).wait()

        for i in range(H):
            gather(i, i)
        for b in range(H):
            gather_wait(b, b)
            scat(b, b)
            gather(b + H, b + H)

        def outer(gi, carry):
            c0 = H + gi * M
            for j in range(M):
                c = c0 + j
                b = (H + j) % M
                q = j
                gather_wait(c, b)
                scat(c, b)
                scat_wait(c - H, q)
                gather(c + H, q)
            return carry
        lax.fori_loop(0, (K - M) // M, outer, 0)

        for j in range(H):
            c = K - H + j
            b = c % M
            gather_wait(c, b)
            scat(c, b)
        for b in range(M):
            scat_wait(0, b)

        plsc.subcore_barrier()
        pltpu.sync_copy(acc.at[pl.ds(sid * RPW, RPW)],
                        out_hbm.at[cid, pl.ds(sid * RPW, RPW)])

    return agg


@functools.partial(
    pl.kernel,
    mesh=_sc_mesh(),
    compiler_params=_SC_PARAMS,
    out_type=jax.ShapeDtypeStruct((2, NP, DW), jnp.float32),
    scratch_types=[
        pltpu.VMEM((K, CHUNK), jnp.int32),
        pltpu.VMEM((CHUNK, DW), jnp.float32),
        pltpu.VMEM_SHARED((NP, DW), jnp.float32),
    ] + [pltpu.SemaphoreType.DMA] * M,
)
def _deg_count(dst_hbm, out_hbm, didx, ones, acc, *sems):
    cid = lax.axis_index("c")
    sid = lax.axis_index("s")
    wid = cid * 16 + sid
    start = wid * K

    pltpu.sync_copy(dst_hbm.at[pl.ds(start, K)], didx)

    def fill(val):
        def body(r, carry):
            ones[r, pl.ds(0, 16)] = jnp.full((16,), val, jnp.float32)
            return carry
        lax.fori_loop(0, CHUNK, body, 0)

    fill(0.0)
    for t in range(RPW // CHUNK):
        pltpu.sync_copy(ones.at[pl.ds(0, CHUNK)],
                        acc.at[pl.ds(sid * RPW + t * CHUNK, CHUNK)])
    plsc.subcore_barrier()
    fill(1.0)

    def scat(c, b):
        pltpu.async_copy(ones, acc.at[didx.at[c]], sems[b], add=True)

    def scat_wait(c, b):
        pltpu.make_async_copy(ones, acc.at[didx.at[c]], sems[b]).wait()

    for b in range(M):
        scat(b, b)

    def body(gi, carry):
        c0 = M + gi * M
        for j in range(M):
            scat_wait(c0 + j - M, j)
            scat(c0 + j, j)
        return carry
    lax.fori_loop(0, (K - M) // M, body, 0)
    for b in range(M):
        scat_wait(K - M + b, b)

    plsc.subcore_barrier()
    pltpu.sync_copy(acc.at[pl.ds(sid * RPW, RPW)],
                    out_hbm.at[cid, pl.ds(sid * RPW, RPW)])


_agg16 = _make_edge_agg(16)
_agg32 = _make_edge_agg(32)


def _tc_prep_body(xp_ref, w_ref, degp_ref, h1p_ref, dinv_ref):
    deg = 1.0 + degp_ref[0, :, 0:1] + degp_ref[1, :, 0:1]
    dinv = lax.rsqrt(deg)
    h1 = jnp.dot(xp_ref[...], w_ref[...], preferred_element_type=jnp.float32)
    h1p_ref[...] = h1 * dinv
    dinv_ref[...] = dinv


_tc_prep = pl.pallas_call(
    _tc_prep_body,
    out_shape=(jax.ShapeDtypeStruct((NP, 16), jnp.float32),
               jax.ShapeDtypeStruct((NP, 1), jnp.float32)),
)


def _tc_act1_body(parts_ref, hp_ref, dinv_ref, b_ref, out_ref):
    dinv = dinv_ref[...]
    agg = (parts_ref[0] + parts_ref[1] + hp_ref[...]) * dinv + b_ref[...]
    out_ref[...] = _selu(agg) * dinv


_tc_act1 = pl.pallas_call(
    _tc_act1_body,
    out_shape=jax.ShapeDtypeStruct((NP, 16), jnp.float32))


def _tc_mid_body(parts_ref, y_ref, dinv_ref, w_ref, b_ref, out_ref):
    dinv = dinv_ref[...]
    agg = (parts_ref[0] + parts_ref[1] + y_ref[...]) * dinv
    z = jnp.dot(agg, w_ref[...], preferred_element_type=jnp.float32) + b_ref[...]
    out_ref[...] = _selu(z) * dinv


def _make_tc_mid(d_out):
    return pl.pallas_call(
        _tc_mid_body,
        out_shape=jax.ShapeDtypeStruct((NP, d_out), jnp.float32))


_tc_mid2 = _make_tc_mid(32)
_tc_mid3 = _make_tc_mid(32)


def _tc_final_body(parts_ref, y_ref, dinv_ref, w_ref, b_ref, batch_ref,
                   lw1_ref, lb1_ref, lw2_ref, lb2_ref, out_ref):
    agg = (parts_ref[0] + parts_ref[1] + y_ref[...]) * dinv_ref[...]
    z = jnp.dot(agg, w_ref[...], preferred_element_type=jnp.float32) + b_ref[...]
    c4 = _selu(z)
    seg = lax.broadcasted_iota(jnp.int32, (G, NP), 0)
    m = (batch_ref[...] == seg).astype(jnp.float32)
    s = jnp.dot(m, c4, preferred_element_type=jnp.float32)
    col = lax.broadcasted_iota(jnp.int32, (G, 48), 1)
    s = jnp.where(col < 36, s, -jnp.inf)
    s = s - jnp.max(s, axis=1, keepdims=True)
    e = jnp.exp(s)
    p = e / jnp.sum(e, axis=1, keepdims=True)
    h = jnp.dot(p, lw1_ref[...], preferred_element_type=jnp.float32) + lb1_ref[...]
    h = jnp.maximum(h, 0.0)
    o = jnp.dot(h, lw2_ref[...], preferred_element_type=jnp.float32) + lb2_ref[...]
    out_ref[...] = jnp.maximum(o, 0.0)


_tc_final = pl.pallas_call(
    _tc_final_body,
    out_shape=jax.ShapeDtypeStruct((G, 16), jnp.float32))


def kernel(x, edge_index, batch, W1, b1, W2, b2, W3, b3, W4, b4,
           LW1, Lb1, LW2, Lb2):
    f32 = jnp.float32
    src = edge_index[0].astype(jnp.int32)
    dst = edge_index[1].astype(jnp.int32)
    pad_e = EC_ROWS * CHUNK - E
    srcp = jnp.concatenate([src, jnp.full((pad_e,), N, jnp.int32)])
    dstp = jnp.concatenate([dst, jnp.full((pad_e,), N, jnp.int32)])
    srcp = srcp.reshape(EC_ROWS, CHUNK)
    dstp = dstp.reshape(EC_ROWS, CHUNK)
    xp = jnp.zeros((NP, 128), f32).at[:N].set(x)
    batchp = jnp.full((1, NP), G, jnp.int32).at[0, :N].set(batch)

    def padw(w, r, c):
        return jnp.zeros((r, c), f32).at[:w.shape[0], :w.shape[1]].set(w)

    W1p = padw(W1, 128, 16)
    W2p = padw(W2, 16, 32)
    W3p = padw(W3, 32, 32)
    W4p = padw(W4, 32, 48)
    b1p = padw(b1[None], 1, 16)
    b2p = padw(b2[None], 1, 32)
    b3p = padw(b3[None], 1, 32)
    b4p = padw(b4[None], 1, 48)
    LW1p = padw(LW1, 48, 96)
    lb1p = padw(Lb1[None], 1, 96)
    LW2p = padw(LW2, 96, 16)
    lb2p = padw(Lb2[None], 1, 16)

    degp = _deg_count(dstp)
    h1p, dinv = _tc_prep(xp, W1p, degp)
    p1 = _agg16(h1p, srcp, dstp)
    y1 = _tc_act1(p1, h1p, dinv, b1p)
    p2 = _agg16(y1, srcp, dstp)
    y2 = _tc_mid2(p2, y1, dinv, W2p, b2p)
    p3 = _agg32(y2, srcp, dstp)
    y3 = _tc_mid3(p3, y2, dinv, W3p, b3p)
    p4 = _agg32(y3, srcp, dstp)
    out = _tc_final(p4, y3, dinv, W4p, b4p, batchp, LW1p, lb1p, LW2p, lb2p)
    return out[:, :10]


# speedup vs baseline: 51.3852x; 1.8501x over previous
import functools

import jax
import jax.numpy as jnp
from jax import lax
from jax.experimental import pallas as pl
from jax.experimental.pallas import tpu as pltpu
from jax.experimental.pallas import tpu_sc as plsc

N = 10000
NP = 10240
E = 320000
G = 64
CHUNK = 128
K = 80
EC_ROWS = 2560
M = 8
H = M // 2
RPW = NP // 16
DW = 16

_SELU_L = 1.0507009873554805
_SELU_A = 1.6732632423543772


def _selu(x):
    return _SELU_L * jnp.where(x > 0, x, _SELU_A * (jnp.exp(x) - 1.0))


def _sc_mesh():
    return plsc.VectorSubcoreMesh(core_axis_name="c", subcore_axis_name="s")


_SC_PARAMS = pltpu.CompilerParams(use_tc_tiling_on_sc=False)


def _make_edge_agg(d):
    assert K % M == 0

    @functools.partial(
        pl.kernel,
        mesh=_sc_mesh(),
        compiler_params=_SC_PARAMS,
        out_type=jax.ShapeDtypeStruct((2, NP, d), jnp.float32),
        scratch_types=[
            pltpu.VMEM((K, CHUNK), jnp.int32),
            pltpu.VMEM((K, CHUNK), jnp.int32),
            pltpu.VMEM((M, CHUNK, d), jnp.float32),
            pltpu.VMEM_SHARED((NP, d), jnp.float32),
        ] + [pltpu.SemaphoreType.DMA] * (2 * M),
    )
    def agg(h_hbm, src_hbm, dst_hbm, out_hbm, sidx, didx, rows, acc, *sems):
        semg = sems[:M]
        sems_ = sems[M:]
        cid = lax.axis_index("c")
        sid = lax.axis_index("s")
        wid = cid * 16 + sid
        start = wid * K

        pltpu.sync_copy(src_hbm.at[pl.ds(start, K)], sidx)
        pltpu.sync_copy(dst_hbm.at[pl.ds(start, K)], didx)

        def zrow(r, carry):
            for jj in range(d // 16):
                rows[0, r, pl.ds(jj * 16, 16)] = jnp.zeros((16,), jnp.float32)
            return carry
        lax.fori_loop(0, CHUNK, zrow, 0)
        for t in range(RPW // CHUNK):
            pltpu.sync_copy(rows.at[0], acc.at[pl.ds(sid * RPW + t * CHUNK, CHUNK)])
        plsc.subcore_barrier()

        def gather(c, b):
            pltpu.async_copy(h_hbm.at[sidx.at[c]], rows.at[b], semg[b])

        def gather_wait(c, b):
            pltpu.make_async_copy(h_hbm.at[sidx.at[c]], rows.at[b], semg[b]).wait()

        def scat(c, b):
            pltpu.async_copy(rows.at[b], acc.at[didx.at[c]], sems_[b], add=True)

        def scat_wait(c, b):
            pltpu.make_async_copy(rows.at[b], acc.at[didx.at[c]], sems_[b]).wait()

        for i in range(H):
            gather(i, i)
        for b in range(H):
            gather_wait(b, b)
            scat(b, b)
            gather(b + H, b + H)

        def outer(gi, carry):
            c0 = H + gi * M
            for j in range(M):
                c = c0 + j
                b = (H + j) % M
                q = j
                gather_wait(c, b)
                scat(c, b)
                scat_wait(c - H, q)
                gather(c + H, q)
            return carry
        lax.fori_loop(0, (K - M) // M, outer, 0)

        for j in range(H):
            c = K - H + j
            b = c % M
            gather_wait(c, b)
            scat(c, b)
        for b in range(M):
            scat_wait(0, b)

        plsc.subcore_barrier()
        pltpu.sync_copy(acc.at[pl.ds(sid * RPW, RPW)],
                        out_hbm.at[cid, pl.ds(sid * RPW, RPW)])

    return agg


@functools.partial(
    pl.kernel,
    mesh=_sc_mesh(),
    compiler_params=_SC_PARAMS,
    out_type=jax.ShapeDtypeStruct((2, NP, DW), jnp.float32),
    scratch_types=[
        pltpu.VMEM((K, CHUNK), jnp.int32),
        pltpu.VMEM((CHUNK, DW), jnp.float32),
        pltpu.VMEM_SHARED((NP, DW), jnp.float32),
    ] + [pltpu.SemaphoreType.DMA] * M,
)
def _deg_count(dst_hbm, out_hbm, didx, ones, acc, *sems):
    cid = lax.axis_index("c")
    sid = lax.axis_index("s")
    wid = cid * 16 + sid
    start = wid * K

    pltpu.sync_copy(dst_hbm.at[pl.ds(start, K)], didx)

    def fill(val):
        def body(r, carry):
            ones[r, pl.ds(0, 16)] = jnp.full((16,), val, jnp.float32)
            return carry
        lax.fori_loop(0, CHUNK, body, 0)

    fill(0.0)
    for t in range(RPW // CHUNK):
        pltpu.sync_copy(ones.at[pl.ds(0, CHUNK)],
                        acc.at[pl.ds(sid * RPW + t * CHUNK, CHUNK)])
    plsc.subcore_barrier()
    fill(1.0)

    def scat(c, b):
        pltpu.async_copy(ones, acc.at[didx.at[c]], sems[b], add=True)

    def scat_wait(c, b):
        pltpu.make_async_copy(ones, acc.at[didx.at[c]], sems[b]).wait()

    for b in range(M):
        scat(b, b)

    def body(gi, carry):
        c0 = M + gi * M
        for j in range(M):
            scat_wait(c0 + j - M, j)
            scat(c0 + j, j)
        return carry
    lax.fori_loop(0, (K - M) // M, body, 0)
    for b in range(M):
        scat_wait(K - M + b, b)

    plsc.subcore_barrier()
    pltpu.sync_copy(acc.at[pl.ds(sid * RPW, RPW)],
                    out_hbm.at[cid, pl.ds(sid * RPW, RPW)])


_agg16 = _make_edge_agg(16)
_agg32 = _make_edge_agg(32)


def _tc_prep_body(xp_ref, w_ref, degp_ref, h1p_ref, dinv_ref):
    deg = 1.0 + degp_ref[0, :, 0:1] + degp_ref[1, :, 0:1]
    dinv = lax.rsqrt(deg)
    h1 = jnp.dot(xp_ref[...], w_ref[...], preferred_element_type=jnp.float32)
    h1p_ref[...] = h1 * dinv
    dinv_ref[...] = dinv


_tc_prep = pl.pallas_call(
    _tc_prep_body,
    out_shape=(jax.ShapeDtypeStruct((NP, 16), jnp.float32),
               jax.ShapeDtypeStruct((NP, 1), jnp.float32)),
)


def _tc_act1_body(parts_ref, hp_ref, dinv_ref, b_ref, out_ref):
    dinv = dinv_ref[...]
    agg = (parts_ref[0] + parts_ref[1] + hp_ref[...]) * dinv + b_ref[...]
    out_ref[...] = _selu(agg) * dinv


_tc_act1 = pl.pallas_call(
    _tc_act1_body,
    out_shape=jax.ShapeDtypeStruct((NP, 16), jnp.float32))


def _tc_mid_body(parts_ref, y_ref, dinv_ref, w_ref, b_ref, out_ref):
    dinv = dinv_ref[...]
    agg = (parts_ref[0] + parts_ref[1] + y_ref[...]) * dinv
    z = jnp.dot(agg, w_ref[...], preferred_element_type=jnp.float32) + b_ref[...]
    out_ref[...] = _selu(z) * dinv


def _make_tc_mid(d_out):
    return pl.pallas_call(
        _tc_mid_body,
        out_shape=jax.ShapeDtypeStruct((NP, d_out), jnp.float32))


_tc_mid2 = _make_tc_mid(32)
_tc_mid3 = _make_tc_mid(32)


def _tc_final_body(parts_ref, y_ref, dinv_ref, w_ref, b_ref, batch_ref,
                   lw1_ref, lb1_ref, lw2_ref, lb2_ref, out_ref):
    agg = (parts_ref[0] + parts_ref[1] + y_ref[...]) * dinv_ref[...]
    z = jnp.dot(agg, w_ref[...], preferred_element_type=jnp.float32) + b_ref[...]
    c4 = _selu(z)
    seg = lax.broadcasted_iota(jnp.int32, (G, NP), 0)
    m = (batch_ref[...] == seg).astype(jnp.float32)
    s = jnp.dot(m, c4, preferred_element_type=jnp.float32)
    col = lax.broadcasted_iota(jnp.int32, (G, 48), 1)
    s = jnp.where(col < 36, s, -jnp.inf)
    s = s - jnp.max(s, axis=1, keepdims=True)
    e = jnp.exp(s)
    p = e / jnp.sum(e, axis=1, keepdims=True)
    h = jnp.dot(p, lw1_ref[...], preferred_element_type=jnp.float32) + lb1_ref[...]
    h = jnp.maximum(h, 0.0)
    o = jnp.dot(h, lw2_ref[...], preferred_element_type=jnp.float32) + lb2_ref[...]
    out_ref[...] = jnp.maximum(o, 0.0)


_tc_final = pl.pallas_call(
    _tc_final_body,
    out_shape=jax.ShapeDtypeStruct((G, 16), jnp.float32))


def kernel(x, edge_index, batch, W1, b1, W2, b2, W3, b3, W4, b4,
           LW1, Lb1, LW2, Lb2):
    f32 = jnp.float32
    src = edge_index[0].astype(jnp.int32)
    dst = edge_index[1].astype(jnp.int32)
    pad_e = EC_ROWS * CHUNK - E
    pad_ix = N + jnp.arange(pad_e, dtype=jnp.int32) % (NP - N)
    srcp = jnp.concatenate([src, pad_ix])
    dstp = jnp.concatenate([dst, pad_ix])
    srcp = srcp.reshape(EC_ROWS, CHUNK)
    dstp = dstp.reshape(EC_ROWS, CHUNK)
    xp = jnp.zeros((NP, 128), f32).at[:N].set(x)
    batchp = jnp.full((1, NP), G, jnp.int32).at[0, :N].set(batch)

    def padw(w, r, c):
        return jnp.zeros((r, c), f32).at[:w.shape[0], :w.shape[1]].set(w)

    W1p = padw(W1, 128, 16)
    W2p = padw(W2, 16, 32)
    W3p = padw(W3, 32, 32)
    W4p = padw(W4, 32, 48)
    b1p = padw(b1[None], 1, 16)
    b2p = padw(b2[None], 1, 32)
    b3p = padw(b3[None], 1, 32)
    b4p = padw(b4[None], 1, 48)
    LW1p = padw(LW1, 48, 96)
    lb1p = padw(Lb1[None], 1, 96)
    LW2p = padw(LW2, 96, 16)
    lb2p = padw(Lb2[None], 1, 16)

    degp = _deg_count(dstp)
    h1p, dinv = _tc_prep(xp, W1p, degp)
    p1 = _agg16(h1p, srcp, dstp)
    y1 = _tc_act1(p1, h1p, dinv, b1p)
    p2 = _agg16(y1, srcp, dstp)
    y2 = _tc_mid2(p2, y1, dinv, W2p, b2p)
    p3 = _agg32(y2, srcp, dstp)
    y3 = _tc_mid3(p3, y2, dinv, W3p, b3p)
    p4 = _agg32(y3, srcp, dstp)
    out = _tc_final(p4, y3, dinv, W4p, b4p, batchp, LW1p, lb1p, LW2p, lb2p)
    return out[:, :10]


# speedup vs baseline: 52.4937x; 1.0216x over previous
import functools

import jax
import jax.numpy as jnp
from jax import lax
from jax.experimental import pallas as pl
from jax.experimental.pallas import tpu as pltpu
from jax.experimental.pallas import tpu_sc as plsc

N = 10000
NP = 10240
E = 320000
G = 64
CHUNK = 128
K = 80
EC_ROWS = 2560
M = 8
H = M // 2
RPW = NP // 16
DW = 16

_SELU_L = 1.0507009873554805
_SELU_A = 1.6732632423543772


def _selu(x):
    return _SELU_L * jnp.where(x > 0, x, _SELU_A * (jnp.exp(x) - 1.0))


def _sc_mesh():
    return plsc.VectorSubcoreMesh(core_axis_name="c", subcore_axis_name="s")


_SC_PARAMS = pltpu.CompilerParams(use_tc_tiling_on_sc=False)


def _make_edge_agg(d):
    assert K % M == 0

    @functools.partial(
        pl.kernel,
        mesh=_sc_mesh(),
        compiler_params=_SC_PARAMS,
        out_type=jax.ShapeDtypeStruct((2, NP, d), jnp.float32),
        scratch_types=[
            pltpu.VMEM((K, CHUNK), jnp.int32),
            pltpu.VMEM((K, CHUNK), jnp.int32),
            pltpu.VMEM((M, CHUNK, d), jnp.float32),
            pltpu.VMEM_SHARED((NP, d), jnp.float32),
        ] + [pltpu.SemaphoreType.DMA] * (2 * M),
    )
    def agg(h_hbm, src_hbm, dst_hbm, out_hbm, sidx, didx, rows, acc, *sems):
        semg = sems[:M]
        sems_ = sems[M:]
        cid = lax.axis_index("c")
        sid = lax.axis_index("s")
        wid = cid * 16 + sid
        start = wid * K

        pltpu.sync_copy(src_hbm.at[pl.ds(start, K)], sidx)
        pltpu.sync_copy(dst_hbm.at[pl.ds(start, K)], didx)

        def zrow(r, carry):
            for jj in range(d // 16):
                rows[0, r, pl.ds(jj * 16, 16)] = jnp.zeros((16,), jnp.float32)
            return carry
        lax.fori_loop(0, CHUNK, zrow, 0)
        for t in range(RPW // CHUNK):
            pltpu.sync_copy(rows.at[0], acc.at[pl.ds(sid * RPW + t * CHUNK, CHUNK)])
        plsc.subcore_barrier()

        def gather(c, b):
            pltpu.async_copy(h_hbm.at[sidx.at[c]], rows.at[b], semg[b])

        def gather_wait(c, b):
            pltpu.make_async_copy(h_hbm.at[sidx.at[c]], rows.at[b], semg[b]).wait()

        def scat(c, b):
            pltpu.async_copy(rows.at[b], acc.at[didx.at[c]], sems_[b], add=True)

        def scat_wait(c, b):
            pltpu.make_async_copy(rows.at[b], acc.at[didx.at[c]], sems_[b]).wait()

        for i in range(H):
            gather(i, i)
        for b in range(H):
            gather_wait(b, b)
            scat(b, b)
            gather(b + H, b + H)

        def outer(gi, carry):
            c0 = H + gi * M
            for j in range(M):
                c = c0 + j
                b = (H + j) % M
                q = j
                gather_wait(c, b)
                scat(c, b)
                scat_wait(c - H, q)
                gather(c + H, q)
            return carry
        lax.fori_loop(0, (K - M) // M, outer, 0)

        for j in range(H):
            c = K - H + j
            b = c % M
            gather_wait(c, b)
            scat(c, b)
        for b in range(M):
            scat_wait(0, b)

        plsc.subcore_barrier()
        pltpu.sync_copy(acc.at[pl.ds(sid * RPW, RPW)],
                        out_hbm.at[cid, pl.ds(sid * RPW, RPW)])

    return agg


@functools.partial(
    pl.kernel,
    mesh=_sc_mesh(),
    compiler_params=_SC_PARAMS,
    out_type=jax.ShapeDtypeStruct((2, NP, DW), jnp.float32),
    scratch_types=[
        pltpu.VMEM((K, CHUNK), jnp.int32),
        pltpu.VMEM((CHUNK, DW), jnp.float32),
        pltpu.VMEM_SHARED((NP, DW), jnp.float32),
    ] + [pltpu.SemaphoreType.DMA] * M,
)
def _deg_count(dst_hbm, out_hbm, didx, ones, acc, *sems):
    cid = lax.axis_index("c")
    sid = lax.axis_index("s")
    wid = cid * 16 + sid
    start = wid * K

    pltpu.sync_copy(dst_hbm.at[pl.ds(start, K)], didx)

    def fill(val):
        def body(r, carry):
            ones[r, pl.ds(0, 16)] = jnp.full((16,), val, jnp.float32)
            return carry
        lax.fori_loop(0, CHUNK, body, 0)

    fill(0.0)
    for t in range(RPW // CHUNK):
        pltpu.sync_copy(ones.at[pl.ds(0, CHUNK)],
                        acc.at[pl.ds(sid * RPW + t * CHUNK, CHUNK)])
    plsc.subcore_barrier()
    fill(1.0)

    def scat(c, b):
        pltpu.async_copy(ones, acc.at[didx.at[c]], sems[b], add=True)

    def scat_wait(c, b):
        pltpu.make_async_copy(ones, acc.at[didx.at[c]], sems[b]).wait()

    for b in range(M):
        scat(b, b)

    def body(gi, carry):
        c0 = M + gi * M
        for j in range(M):
            scat_wait(c0 + j - M, j)
            scat(c0 + j, j)
        return carry
    lax.fori_loop(0, (K - M) // M, body, 0)
    for b in range(M):
        scat_wait(K - M + b, b)

    plsc.subcore_barrier()
    pltpu.sync_copy(acc.at[pl.ds(sid * RPW, RPW)],
                    out_hbm.at[cid, pl.ds(sid * RPW, RPW)])


_agg16 = _make_edge_agg(16)
_agg32 = _make_edge_agg(32)


def _col_is(d, c):
    return lax.broadcasted_iota(jnp.int32, (NP, d), 1) == c


def _tc_prep_body(xp_ref, w_ref, degp_ref, h1p_ref):
    deg = 1.0 + degp_ref[0, :, 0:1] + degp_ref[1, :, 0:1]
    dinv = lax.rsqrt(deg)
    h1 = jnp.dot(xp_ref[...], w_ref[...], preferred_element_type=jnp.float32)
    h1p_ref[...] = jnp.where(_col_is(16, 15), dinv, h1 * dinv)


_tc_prep = pl.pallas_call(
    _tc_prep_body,
    out_shape=jax.ShapeDtypeStruct((NP, 16), jnp.float32),
)


def _tc_act1_body(parts_ref, hp_ref, b_ref, out_ref):
    hp = hp_ref[...]
    dinv = hp[:, 15:16]
    agg = (parts_ref[0] + parts_ref[1] + hp) * dinv + b_ref[...]
    out_ref[...] = jnp.where(_col_is(16, 15), dinv, _selu(agg) * dinv)


_tc_act1 = pl.pallas_call(
    _tc_act1_body,
    out_shape=jax.ShapeDtypeStruct((NP, 16), jnp.float32))


def _tc_mid_body(dcol, parts_ref, y_ref, w_ref, b_ref, out_ref):
    y = y_ref[...]
    dinv = y[:, dcol:dcol + 1]
    agg = (parts_ref[0] + parts_ref[1] + y) * dinv
    z = jnp.dot(agg, w_ref[...], preferred_element_type=jnp.float32) + b_ref[...]
    out_ref[...] = jnp.where(_col_is(32, 31), dinv, _selu(z) * dinv)


def _make_tc_mid(dcol):
    return pl.pallas_call(
        functools.partial(_tc_mid_body, dcol),
        out_shape=jax.ShapeDtypeStruct((NP, 32), jnp.float32))


_tc_mid2 = _make_tc_mid(15)
_tc_mid3 = _make_tc_mid(31)


def _tc_final_body(parts_ref, y_ref, w_ref, b_ref, batch_ref,
                   lw1_ref, lb1_ref, lw2_ref, lb2_ref, out_ref):
    y = y_ref[...]
    agg = (parts_ref[0] + parts_ref[1] + y) * y[:, 31:32]
    z = jnp.dot(agg, w_ref[...], preferred_element_type=jnp.float32) + b_ref[...]
    c4 = _selu(z)
    seg = lax.broadcasted_iota(jnp.int32, (G, NP), 0)
    m = (batch_ref[...] == seg).astype(jnp.float32)
    s = jnp.dot(m, c4, preferred_element_type=jnp.float32)
    col = lax.broadcasted_iota(jnp.int32, (G, 48), 1)
    s = jnp.where(col < 36, s, -jnp.inf)
    s = s - jnp.max(s, axis=1, keepdims=True)
    e = jnp.exp(s)
    p = e / jnp.sum(e, axis=1, keepdims=True)
    h = jnp.dot(p, lw1_ref[...], preferred_element_type=jnp.float32) + lb1_ref[...]
    h = jnp.maximum(h, 0.0)
    o = jnp.dot(h, lw2_ref[...], preferred_element_type=jnp.float32) + lb2_ref[...]
    out_ref[...] = jnp.maximum(o, 0.0)


_tc_final = pl.pallas_call(
    _tc_final_body,
    out_shape=jax.ShapeDtypeStruct((G, 16), jnp.float32))


def kernel(x, edge_index, batch, W1, b1, W2, b2, W3, b3, W4, b4,
           LW1, Lb1, LW2, Lb2):
    f32 = jnp.float32
    src = edge_index[0].astype(jnp.int32)
    dst = edge_index[1].astype(jnp.int32)
    pad_e = EC_ROWS * CHUNK - E
    pad_ix = N + jnp.arange(pad_e, dtype=jnp.int32) % (NP - N)
    srcp = jnp.concatenate([src, pad_ix])
    dstp = jnp.concatenate([dst, pad_ix])
    srcp = srcp.reshape(EC_ROWS, CHUNK)
    dstp = dstp.reshape(EC_ROWS, CHUNK)
    xp = jnp.zeros((NP, 128), f32).at[:N].set(x)
    batchp = jnp.full((1, NP), G, jnp.int32).at[0, :N].set(batch)

    def padw(w, r, c):
        return jnp.zeros((r, c), f32).at[:w.shape[0], :w.shape[1]].set(w)

    W1p = padw(W1, 128, 16)
    W2p = padw(W2, 16, 32)
    W3p = padw(W3, 32, 32)
    W4p = padw(W4, 32, 48)
    b1p = padw(b1[None], 1, 16)
    b2p = padw(b2[None], 1, 32)
    b3p = padw(b3[None], 1, 32)
    b4p = padw(b4[None], 1, 48)
    LW1p = padw(LW1, 48, 96)
    lb1p = padw(Lb1[None], 1, 96)
    LW2p = padw(LW2, 96, 16)
    lb2p = padw(Lb2[None], 1, 16)

    degp = _deg_count(dstp)
    h1p = _tc_prep(xp, W1p, degp)
    p1 = _agg16(h1p, srcp, dstp)
    y1 = _tc_act1(p1, h1p, b1p)
    p2 = _agg16(y1, srcp, dstp)
    y2 = _tc_mid2(p2, y1, W2p, b2p)
    p3 = _agg32(y2, srcp, dstp)
    y3 = _tc_mid3(p3, y2, W3p, b3p)
    p4 = _agg32(y3, srcp, dstp)
    out = _tc_final(p4, y3, W4p, b4p, batchp, LW1p, lb1p, LW2p, lb2p)
    return out[:, :10]


# speedup vs baseline: 54.4219x; 1.0367x over previous
import functools

import jax
import jax.numpy as jnp
from jax import lax
from jax.experimental import pallas as pl
from jax.experimental.pallas import tpu as pltpu
from jax.experimental.pallas import tpu_sc as plsc

N = 10000
NP = 10240
E = 320000
G = 64
CHUNK = 128
K = 80
EC_ROWS = 2560
M = 8
H = M // 2
RPW = NP // 16
DW = 16

_SELU_L = 1.0507009873554805
_SELU_A = 1.6732632423543772


def _selu(x):
    return _SELU_L * jnp.where(x > 0, x, _SELU_A * (jnp.exp(x) - 1.0))


def _sc_mesh():
    return plsc.VectorSubcoreMesh(core_axis_name="c", subcore_axis_name="s")


_SC_PARAMS = pltpu.CompilerParams(use_tc_tiling_on_sc=False)


def _make_edge_agg(d):
    assert K % M == 0

    @functools.partial(
        pl.kernel,
        mesh=_sc_mesh(),
        compiler_params=_SC_PARAMS,
        out_type=jax.ShapeDtypeStruct((2, NP, d), jnp.float32),
        scratch_types=[
            pltpu.VMEM((K, CHUNK), jnp.int32),
            pltpu.VMEM((K, CHUNK), jnp.int32),
            pltpu.VMEM((M, CHUNK, d), jnp.float32),
            pltpu.VMEM_SHARED((NP, d), jnp.float32),
        ] + [pltpu.SemaphoreType.DMA] * (2 * M),
    )
    def agg(h_hbm, src_hbm, dst_hbm, out_hbm, sidx, didx, rows, acc, *sems):
        semg = sems[:M]
        sems_ = sems[M:]
        cid = lax.axis_index("c")
        sid = lax.axis_index("s")
        wid = cid * 16 + sid
        start = wid * K

        pltpu.async_copy(src_hbm.at[pl.ds(start, K)], sidx, semg[0])
        pltpu.async_copy(dst_hbm.at[pl.ds(start, K)], didx, semg[1])

        def zrow(r, carry):
            for jj in range(d // 16):
                rows[0, r, pl.ds(jj * 16, 16)] = jnp.zeros((16,), jnp.float32)
            return carry
        lax.fori_loop(0, CHUNK, zrow, 0)
        for t in range(RPW // CHUNK):
            pltpu.async_copy(rows.at[0],
                             acc.at[pl.ds(sid * RPW + t * CHUNK, CHUNK)],
                             sems_[t])
        pltpu.make_async_copy(src_hbm.at[pl.ds(start, K)], sidx, semg[0]).wait()
        pltpu.make_async_copy(dst_hbm.at[pl.ds(start, K)], didx, semg[1]).wait()
        for t in range(RPW // CHUNK):
            pltpu.make_async_copy(
                rows.at[0], acc.at[pl.ds(sid * RPW + t * CHUNK, CHUNK)],
                sems_[t]).wait()
        plsc.subcore_barrier()

        def gather(c, b):
            pltpu.async_copy(h_hbm.at[sidx.at[c]], rows.at[b], semg[b])

        def gather_wait(c, b):
            pltpu.make_async_copy(h_hbm.at[sidx.at[c]], rows.at[b], semg[b]).wait()

        def scat(c, b):
            pltpu.async_copy(rows.at[b], acc.at[didx.at[c]], sems_[b], add=True)

        def scat_wait(c, b):
            pltpu.make_async_copy(rows.at[b], acc.at[didx.at[c]], sems_[b]).wait()

        for i in range(H):
            gather(i, i)
        for b in range(H):
            gather_wait(b, b)
            scat(b, b)
            gather(b + H, b + H)

        def outer(gi, carry):
            c0 = H + gi * M
            for j in range(M):
                c = c0 + j
                b = (H + j) % M
                q = j
                gather_wait(c, b)
                scat(c, b)
                scat_wait(c - H, q)
                gather(c + H, q)
            return carry
        lax.fori_loop(0, (K - M) // M, outer, 0)

        for j in range(H):
            c = K - H + j
            b = c % M
            gather_wait(c, b)
            scat(c, b)
        for b in range(M):
            scat_wait(0, b)

        plsc.subcore_barrier()
        pltpu.sync_copy(acc.at[pl.ds(sid * RPW, RPW)],
                        out_hbm.at[cid, pl.ds(sid * RPW, RPW)])

    return agg


@functools.partial(
    pl.kernel,
    mesh=_sc_mesh(),
    compiler_params=_SC_PARAMS,
    out_type=jax.ShapeDtypeStruct((2, NP, DW), jnp.float32),
    scratch_types=[
        pltpu.VMEM((K, CHUNK), jnp.int32),
        pltpu.VMEM((CHUNK, DW), jnp.float32),
        pltpu.VMEM_SHARED((NP, DW), jnp.float32),
    ] + [pltpu.SemaphoreType.DMA] * M,
)
def _deg_count(dst_hbm, out_hbm, didx, ones, acc, *sems):
    cid = lax.axis_index("c")
    sid = lax.axis_index("s")
    wid = cid * 16 + sid
    start = wid * K

    pltpu.async_copy(dst_hbm.at[pl.ds(start, K)], didx, sems[M - 1])

    def fill(val):
        def body(r, carry):
            ones[r, pl.ds(0, 16)] = jnp.full((16,), val, jnp.float32)
            return carry
        lax.fori_loop(0, CHUNK, body, 0)

    fill(0.0)
    for t in range(RPW // CHUNK):
        pltpu.async_copy(ones.at[pl.ds(0, CHUNK)],
                         acc.at[pl.ds(sid * RPW + t * CHUNK, CHUNK)],
                         sems[t])
    for t in range(RPW // CHUNK):
        pltpu.make_async_copy(
            ones.at[pl.ds(0, CHUNK)],
            acc.at[pl.ds(sid * RPW + t * CHUNK, CHUNK)], sems[t]).wait()
    fill(1.0)
    pltpu.make_async_copy(dst_hbm.at[pl.ds(start, K)], didx, sems[M - 1]).wait()
    plsc.subcore_barrier()

    def scat(c, b):
        pltpu.async_copy(ones, acc.at[didx.at[c]], sems[b], add=True)

    def scat_wait(c, b):
        pltpu.make_async_copy(ones, acc.at[didx.at[c]], sems[b]).wait()

    for b in range(M):
        scat(b, b)

    def body(gi, carry):
        c0 = M + gi * M
        for j in range(M):
            scat_wait(c0 + j - M, j)
            scat(c0 + j, j)
        return carry
    lax.fori_loop(0, (K - M) // M, body, 0)
    for b in range(M):
        scat_wait(K - M + b, b)

    plsc.subcore_barrier()
    pltpu.sync_copy(acc.at[pl.ds(sid * RPW, RPW)],
                    out_hbm.at[cid, pl.ds(sid * RPW, RPW)])


_agg16 = _make_edge_agg(16)
_agg32 = _make_edge_agg(32)


def _col_is(d, c):
    return lax.broadcasted_iota(jnp.int32, (NP, d), 1) == c


def _tc_prep_body(xp_ref, w_ref, degp_ref, h1p_ref):
    deg = 1.0 + degp_ref[0, :, 0:1] + degp_ref[1, :, 0:1]
    dinv = lax.rsqrt(deg)
    h1 = jnp.dot(xp_ref[...], w_ref[...], preferred_element_type=jnp.float32)
    h1p_ref[...] = jnp.where(_col_is(16, 15), dinv, h1 * dinv)


_tc_prep = pl.pallas_call(
    _tc_prep_body,
    out_shape=jax.ShapeDtypeStruct((NP, 16), jnp.float32),
)


def _tc_act1_body(parts_ref, hp_ref, b_ref, out_ref):
    hp = hp_ref[...]
    dinv = hp[:, 15:16]
    agg = (parts_ref[0] + parts_ref[1] + hp) * dinv + b_ref[...]
    out_ref[...] = jnp.where(_col_is(16, 15), dinv, _selu(agg) * dinv)


_tc_act1 = pl.pallas_call(
    _tc_act1_body,
    out_shape=jax.ShapeDtypeStruct((NP, 16), jnp.float32))


def _tc_mid_body(dcol, parts_ref, y_ref, w_ref, b_ref, out_ref):
    y = y_ref[...]
    dinv = y[:, dcol:dcol + 1]
    agg = (parts_ref[0] + parts_ref[1] + y) * dinv
    z = jnp.dot(agg, w_ref[...], preferred_element_type=jnp.float32) + b_ref[...]
    out_ref[...] = jnp.where(_col_is(32, 31), dinv, _selu(z) * dinv)


def _make_tc_mid(dcol):
    return pl.pallas_call(
        functools.partial(_tc_mid_body, dcol),
        out_shape=jax.ShapeDtypeStruct((NP, 32), jnp.float32))


_tc_mid2 = _make_tc_mid(15)
_tc_mid3 = _make_tc_mid(31)


def _tc_final_body(parts_ref, y_ref, w_ref, b_ref, batch_ref,
                   lw1_ref, lb1_ref, lw2_ref, lb2_ref, out_ref):
    y = y_ref[...]
    agg = (parts_ref[0] + parts_ref[1] + y) * y[:, 31:32]
    z = jnp.dot(agg, w_ref[...], preferred_element_type=jnp.float32) + b_ref[...]
    c4 = _selu(z)
    seg = lax.broadcasted_iota(jnp.int32, (G, NP), 0)
    m = (batch_ref[...] == seg).astype(jnp.float32)
    s = jnp.dot(m, c4, preferred_element_type=jnp.float32)
    col = lax.broadcasted_iota(jnp.int32, (G, 48), 1)
    s = jnp.where(col < 36, s, -jnp.inf)
    s = s - jnp.max(s, axis=1, keepdims=True)
    e = jnp.exp(s)
    p = e / jnp.sum(e, axis=1, keepdims=True)
    h = jnp.dot(p, lw1_ref[...], preferred_element_type=jnp.float32) + lb1_ref[...]
    h = jnp.maximum(h, 0.0)
    o = jnp.dot(h, lw2_ref[...], preferred_element_type=jnp.float32) + lb2_ref[...]
    out_ref[...] = jnp.maximum(o, 0.0)


_tc_final = pl.pallas_call(
    _tc_final_body,
    out_shape=jax.ShapeDtypeStruct((G, 16), jnp.float32))


def kernel(x, edge_index, batch, W1, b1, W2, b2, W3, b3, W4, b4,
           LW1, Lb1, LW2, Lb2):
    f32 = jnp.float32
    src = edge_index[0].astype(jnp.int32)
    dst = edge_index[1].astype(jnp.int32)
    pad_e = EC_ROWS * CHUNK - E
    pad_ix = N + jnp.arange(pad_e, dtype=jnp.int32) % (NP - N)
    srcp = jnp.concatenate([src, pad_ix])
    dstp = jnp.concatenate([dst, pad_ix])
    srcp = srcp.reshape(EC_ROWS, CHUNK)
    dstp = dstp.reshape(EC_ROWS, CHUNK)
    xp = jnp.zeros((NP, 128), f32).at[:N].set(x)
    batchp = jnp.full((1, NP), G, jnp.int32).at[0, :N].set(batch)

    def padw(w, r, c):
        return jnp.zeros((r, c), f32).at[:w.shape[0], :w.shape[1]].set(w)

    W1p = padw(W1, 128, 16)
    W2p = padw(W2, 16, 32)
    W3p = padw(W3, 32, 32)
    W4p = padw(W4, 32, 48)
    b1p = padw(b1[None], 1, 16)
    b2p = padw(b2[None], 1, 32)
    b3p = padw(b3[None], 1, 32)
    b4p = padw(b4[None], 1, 48)
    LW1p = padw(LW1, 48, 96)
    lb1p = padw(Lb1[None], 1, 96)
    LW2p = padw(LW2, 96, 16)
    lb2p = padw(Lb2[None], 1, 16)

    degp = _deg_count(dstp)
    h1p = _tc_prep(xp, W1p, degp)
    p1 = _agg16(h1p, srcp, dstp)
    y1 = _tc_act1(p1, h1p, b1p)
    p2 = _agg16(y1, srcp, dstp)
    y2 = _tc_mid2(p2, y1, W2p, b2p)
    p3 = _agg32(y2, srcp, dstp)
    y3 = _tc_mid3(p3, y2, W3p, b3p)
    p4 = _agg32(y3, srcp, dstp)
    out = _tc_final(p4, y3, W4p, b4p, batchp, LW1p, lb1p, LW2p, lb2p)
    return out[:, :10]
